# Initial kernel scaffold; baseline (speedup 1.0000x reference)
import jax
import jax.numpy as jnp
from jax.experimental import pallas as pl


def kernel(x, edge_index, edge_vals, W1, b1, scalar, Wout, bout):
    raise NotImplementedError("write your pallas kernel here")


# speedup vs baseline: 4.0942x; 4.0942x over previous
import functools

import jax
import jax.numpy as jnp
from jax import lax
from jax.experimental import pallas as pl
from jax.experimental.pallas import tpu as pltpu
from jax.experimental.pallas import tpu_sc as plsc

N = 10000
F = 128
E = 320000

NC = 2
NS = 16
NW = NC * NS
EPW = E // NW
K = 80
NCHUNK = EPW // K
ROW_STRIDE = 624
ROW_COPY = 640


def _spmm_partials(h, dst, src, vals, zeros):
    mesh = plsc.VectorSubcoreMesh(core_axis_name="c", subcore_axis_name="s")

    @functools.partial(
        pl.kernel,
        out_type=jax.ShapeDtypeStruct((NC, N, F), jnp.float32),
        mesh=mesh,
        scratch_types=[
            pltpu.VMEM((K,), jnp.int32),
            pltpu.VMEM((K,), jnp.int32),
            pltpu.VMEM((K,), jnp.float32),
            pltpu.VMEM((K, F), jnp.float32),
            pltpu.VMEM_SHARED((N, F), jnp.float32),
            pltpu.SemaphoreType.DMA,
        ],
    )
    def k(h_hbm, dst_hbm, src_hbm, vals_hbm, z_hbm, out_hbm,
          src_v, dst_v, vals_v, rows_v, acc_sh, sem):
        cid = lax.axis_index("c")
        sid = lax.axis_index("s")
        wid = cid * NS + sid

        rstart = pl.multiple_of(sid * ROW_STRIDE, 8)

        pltpu.sync_copy(z_hbm.at[pl.ds(rstart, ROW_COPY)],
                        acc_sh.at[pl.ds(rstart, ROW_COPY)])
        plsc.subcore_barrier()

        def chunk(ci, carry):
            base = pl.multiple_of(wid * EPW + ci * K, 8)
            pltpu.sync_copy(src_hbm.at[pl.ds(base, K)], src_v)
            pltpu.sync_copy(dst_hbm.at[pl.ds(base, K)], dst_v)
            pltpu.sync_copy(vals_hbm.at[pl.ds(base, K)], vals_v)
            pltpu.async_copy(h_hbm.at[src_v], rows_v, sem).wait()

            def scale(g, c2):
                vvec = vals_v[pl.ds(16 * g, 16)]
                for i in range(16):
                    v = vvec[i]
                    e = 16 * g + i
                    for j in range(F // 16):
                        sl = pl.ds(16 * j, 16)
                        rows_v[e, sl] = rows_v[e, sl] * v
                return c2
            lax.fori_loop(0, K // 16, scale, 0)

            pltpu.sync_copy(rows_v, acc_sh.at[dst_v], add=True)
            return carry
        lax.fori_loop(0, NCHUNK, chunk, 0)

        plsc.subcore_barrier()
        pltpu.sync_copy(acc_sh.at[pl.ds(rstart, ROW_COPY)],
                        out_hbm.at[cid, pl.ds(rstart, ROW_COPY)])

    return k(h, dst, src, vals, zeros)


_BM = 1000


def _mm_in(x, w_t, b):
    def body(x_ref, w_ref, b_ref, o_ref):
        o_ref[...] = jnp.dot(x_ref[...], w_ref[...],
                             preferred_element_type=jnp.float32) + b_ref[...]
    return pl.pallas_call(
        body,
        grid=(N // _BM,),
        in_specs=[pl.BlockSpec((_BM, F), lambda i: (i, 0)),
                  pl.BlockSpec((F, F), lambda i: (0, 0)),
                  pl.BlockSpec((1, F), lambda i: (0, 0))],
        out_specs=pl.BlockSpec((_BM, F), lambda i: (i, 0)),
        out_shape=jax.ShapeDtypeStruct((N, F), jnp.float32),
    )(x, w_t, b.reshape(1, F))


def _combine_scale(parts, scal):
    def body(s_ref, p_ref, o_ref):
        s = p_ref[0] + p_ref[1]
        o_ref[...] = jnp.where(s > 0, s, (jnp.exp(s) - 1.0)) * s_ref[0]
    return pl.pallas_call(
        body,
        grid=(N // _BM,),
        in_specs=[pl.BlockSpec(memory_space=pltpu.SMEM),
                  pl.BlockSpec((NC, _BM, F), lambda i: (0, i, 0))],
        out_specs=pl.BlockSpec((_BM, F), lambda i: (i, 0)),
        out_shape=jax.ShapeDtypeStruct((N, F), jnp.float32),
    )(scal, parts)


def _combine_mm_out(parts, w_t, b):
    def body(p_ref, w_ref, b_ref, o_ref):
        s = p_ref[0] + p_ref[1]
        h = jnp.where(s > 0, s, (jnp.exp(s) - 1.0))
        o_ref[...] = jnp.dot(h, w_ref[...],
                             preferred_element_type=jnp.float32) + b_ref[...]
    return pl.pallas_call(
        body,
        grid=(N // _BM,),
        in_specs=[pl.BlockSpec((NC, _BM, F), lambda i: (0, i, 0)),
                  pl.BlockSpec((F, F), lambda i: (0, 0)),
                  pl.BlockSpec((1, F), lambda i: (0, 0))],
        out_specs=pl.BlockSpec((_BM, F), lambda i: (i, 0)),
        out_shape=jax.ShapeDtypeStruct((N, F), jnp.float32),
    )(parts, w_t, b.reshape(1, F))


def kernel(x, edge_index, edge_vals, W1, b1, scalar, Wout, bout):
    dst = edge_index[0]
    src = edge_index[1]
    zeros = jnp.zeros((N, F), jnp.float32)

    h = _mm_in(x, W1.T, b1)
    parts = _spmm_partials(h, dst, src, edge_vals, zeros)
    for _ in range(2):
        g = _combine_scale(parts, scalar)
        parts = _spmm_partials(g, dst, src, edge_vals, zeros)
    return _combine_mm_out(parts, Wout.T, bout)


# speedup vs baseline: 8.3696x; 2.0443x over previous
import functools

import jax
import jax.numpy as jnp
from jax import lax
from jax.experimental import pallas as pl
from jax.experimental.pallas import tpu as pltpu
from jax.experimental.pallas import tpu_sc as plsc

N = 10000
F = 128
E = 320000

NC = 2
NS = 16
NW = NC * NS
EPW = E // NW
K = 80
NCHUNK = EPW // K
ROW_STRIDE = 624
ROW_COPY = 640


def _spmm_partials(h, dst3, src3, vals3, zeros):
    mesh = plsc.VectorSubcoreMesh(core_axis_name="c", subcore_axis_name="s")

    @functools.partial(
        pl.kernel,
        out_type=jax.ShapeDtypeStruct((NC, N, F), jnp.float32),
        mesh=mesh,
        scratch_types=[
            pltpu.VMEM((2, K), jnp.int32),
            pltpu.VMEM((2, K), jnp.int32),
            pltpu.VMEM((2, K), jnp.float32),
            pltpu.VMEM((K, F), jnp.float32),
            pltpu.VMEM((K, F), jnp.float32),
            pltpu.VMEM_SHARED((N, F), jnp.float32),
            pltpu.SemaphoreType.DMA,
            pltpu.SemaphoreType.DMA,
            pltpu.SemaphoreType.DMA,
            pltpu.SemaphoreType.DMA,
        ],
    )
    def k(h_hbm, dst_hbm, src_hbm, vals_hbm, z_hbm, out_hbm,
          src_v, dst_v, vals_v, rows0_v, rows1_v, acc_sh,
          gsem0, gsem1, isem0, isem1):
        cid = lax.axis_index("c")
        sid = lax.axis_index("s")
        wid = cid * NS + sid
        rows = (rows0_v, rows1_v)
        gsems = (gsem0, gsem1)
        isems = (isem0, isem1)

        rstart = pl.multiple_of(sid * ROW_STRIDE, 8)

        def start_idx(ci, b):
            pltpu.async_copy(src_hbm.at[wid, ci], src_v.at[b], isems[b])
            pltpu.async_copy(dst_hbm.at[wid, ci], dst_v.at[b], isems[b])
            pltpu.async_copy(vals_hbm.at[wid, ci], vals_v.at[b], isems[b])

        def wait_idx(b):
            pltpu.make_async_copy(src_hbm.at[0, 0], src_v.at[b],
                                  isems[b]).wait()
            pltpu.make_async_copy(dst_hbm.at[0, 0], dst_v.at[b],
                                  isems[b]).wait()
            pltpu.make_async_copy(vals_hbm.at[0, 0], vals_v.at[b],
                                  isems[b]).wait()

        def start_gather(b):
            pltpu.async_copy(h_hbm.at[src_v.at[b]], rows[b], gsems[b])

        def compute(b):
            pltpu.make_async_copy(h_hbm.at[pl.ds(0, K)], rows[b],
                                  gsems[b]).wait()
            rv = rows[b]

            def scale(g, c2):
                vvec = vals_v[b, pl.ds(16 * g, 16)]
                for i in range(16):
                    v = vvec[i]
                    e = 16 * g + i
                    for j in range(F // 16):
                        sl = pl.ds(16 * j, 16)
                        rv[e, sl] = rv[e, sl] * v
                return c2
            lax.fori_loop(0, K // 16, scale, 0)
            pltpu.sync_copy(rv, acc_sh.at[dst_v.at[b]], add=True)

        start_idx(0, 0)
        start_idx(1, 1)
        pltpu.sync_copy(z_hbm.at[pl.ds(rstart, ROW_COPY)],
                        acc_sh.at[pl.ds(rstart, ROW_COPY)])
        plsc.subcore_barrier()
        wait_idx(0)
        start_gather(0)

        def pair(cc, carry):
            ci = 2 * cc
            wait_idx(1)
            start_gather(1)
            compute(0)
            start_idx(ci + 2, 0)
            wait_idx(0)
            start_gather(0)
            compute(1)
            start_idx(jnp.minimum(ci + 3, NCHUNK - 1), 1)
            return carry
        lax.fori_loop(0, (NCHUNK - 1) // 2, pair, 0)
        compute(0)
        wait_idx(1)

        plsc.subcore_barrier()
        pltpu.sync_copy(acc_sh.at[pl.ds(rstart, ROW_COPY)],
                        out_hbm.at[cid, pl.ds(rstart, ROW_COPY)])

    return k(h, dst3, src3, vals3, zeros)


_BM = 1000


def _mm_in(x, w_t, b):
    def body(x_ref, w_ref, b_ref, o_ref):
        o_ref[...] = jnp.dot(x_ref[...], w_ref[...],
                             preferred_element_type=jnp.float32) + b_ref[...]
    return pl.pallas_call(
        body,
        grid=(N // _BM,),
        in_specs=[pl.BlockSpec((_BM, F), lambda i: (i, 0)),
                  pl.BlockSpec((F, F), lambda i: (0, 0)),
                  pl.BlockSpec((1, F), lambda i: (0, 0))],
        out_specs=pl.BlockSpec((_BM, F), lambda i: (i, 0)),
        out_shape=jax.ShapeDtypeStruct((N, F), jnp.float32),
    )(x, w_t, b.reshape(1, F))


def _combine_scale(parts, scal):
    def body(s_ref, p_ref, o_ref):
        s = p_ref[0] + p_ref[1]
        o_ref[...] = jnp.where(s > 0, s, (jnp.exp(s) - 1.0)) * s_ref[0]
    return pl.pallas_call(
        body,
        grid=(N // _BM,),
        in_specs=[pl.BlockSpec(memory_space=pltpu.SMEM),
                  pl.BlockSpec((NC, _BM, F), lambda i: (0, i, 0))],
        out_specs=pl.BlockSpec((_BM, F), lambda i: (i, 0)),
        out_shape=jax.ShapeDtypeStruct((N, F), jnp.float32),
    )(scal, parts)


def _combine_mm_out(parts, w_t, b):
    def body(p_ref, w_ref, b_ref, o_ref):
        s = p_ref[0] + p_ref[1]
        h = jnp.where(s > 0, s, (jnp.exp(s) - 1.0))
        o_ref[...] = jnp.dot(h, w_ref[...],
                             preferred_element_type=jnp.float32) + b_ref[...]
    return pl.pallas_call(
        body,
        grid=(N // _BM,),
        in_specs=[pl.BlockSpec((NC, _BM, F), lambda i: (0, i, 0)),
                  pl.BlockSpec((F, F), lambda i: (0, 0)),
                  pl.BlockSpec((1, F), lambda i: (0, 0))],
        out_specs=pl.BlockSpec((_BM, F), lambda i: (i, 0)),
        out_shape=jax.ShapeDtypeStruct((N, F), jnp.float32),
    )(parts, w_t, b.reshape(1, F))


def kernel(x, edge_index, edge_vals, W1, b1, scalar, Wout, bout):
    dst3 = edge_index[0].reshape(NW, NCHUNK, K)
    src3 = edge_index[1].reshape(NW, NCHUNK, K)
    vals3 = edge_vals.reshape(NW, NCHUNK, K)
    zeros = jnp.zeros((N, F), jnp.float32)

    h = _mm_in(x, W1.T, b1)
    parts = _spmm_partials(h, dst3, src3, vals3, zeros)
    for _ in range(2):
        g = _combine_scale(parts, scalar)
        parts = _spmm_partials(g, dst3, src3, vals3, zeros)
    return _combine_mm_out(parts, Wout.T, bout)


# speedup vs baseline: 9.7369x; 1.1634x over previous
import functools

import jax
import jax.numpy as jnp
from jax import lax
from jax.experimental import pallas as pl
from jax.experimental.pallas import tpu as pltpu
from jax.experimental.pallas import tpu_sc as plsc

N = 10000
F = 128
E = 320000

NC = 2
NS = 16
NW = NC * NS
EPW = E // NW
K = 80
NCHUNK = EPW // K
ROW_STRIDE = 624
ROW_COPY = 640


def _spmm_partials(h, packed, vals3, zeros):
    mesh = plsc.VectorSubcoreMesh(core_axis_name="c", subcore_axis_name="s")

    @functools.partial(
        pl.kernel,
        out_type=jax.ShapeDtypeStruct((NC, N, F), jnp.float32),
        mesh=mesh,
        scratch_types=[
            pltpu.VMEM((4, 2, K), jnp.int32),
            pltpu.VMEM((4, K), jnp.float32),
            pltpu.VMEM((K, F), jnp.float32),
            pltpu.VMEM((K, F), jnp.float32),
            pltpu.VMEM_SHARED((N, F), jnp.float32),
            [pltpu.SemaphoreType.DMA] * 4,
            [pltpu.SemaphoreType.DMA] * 2,
            [pltpu.SemaphoreType.DMA] * 2,
        ],
    )
    def k(h_hbm, e_hbm, v_hbm, z_hbm, out_hbm,
          pbuf, vbuf, rows0_v, rows1_v, acc_sh, isems, gsems, ssems):
        cid = lax.axis_index("c")
        sid = lax.axis_index("s")
        wid = cid * NS + sid
        rows = (rows0_v, rows1_v)

        rstart = pl.multiple_of(sid * ROW_STRIDE, 8)

        def start_idx(ci, q):
            pltpu.async_copy(e_hbm.at[wid, ci], pbuf.at[q], isems[q])
            pltpu.async_copy(v_hbm.at[wid, ci], vbuf.at[q], isems[q])

        def wait_idx(q):
            pltpu.make_async_copy(e_hbm.at[0, 0], pbuf.at[q],
                                  isems[q]).wait()
            pltpu.make_async_copy(v_hbm.at[0, 0], vbuf.at[q],
                                  isems[q]).wait()

        def start_gather(q, b):
            pltpu.async_copy(h_hbm.at[pbuf.at[q, 0]], rows[b], gsems[b])

        def wait_rows_bytes(sem, b):
            pltpu.make_async_copy(h_hbm.at[pl.ds(0, K)], rows[b],
                                  sem).wait()

        def do_chunk(ci, q, wait_prev_scatter=True, issue_next=True):
            b = q % 2
            qn, bn, q2 = (q + 1) % 4, (b + 1) % 2, (q + 2) % 4
            if issue_next:
                wait_idx(qn)
                if wait_prev_scatter:
                    wait_rows_bytes(ssems[bn], bn)
                start_gather(qn, bn)
            wait_rows_bytes(gsems[b], b)
            rv = rows[b]

            def scale(g, c2):
                vvec = vbuf[q, pl.ds(16 * g, 16)]
                for i in range(16):
                    v = vvec[i]
                    e = 16 * g + i
                    for j in range(F // 16):
                        sl = pl.ds(16 * j, 16)
                        rv[e, sl] = rv[e, sl] * v
                return c2
            lax.fori_loop(0, K // 16, scale, 0)
            pltpu.async_copy(rv, acc_sh.at[pbuf.at[q, 1]], ssems[b],
                             add=True)
            if issue_next:
                start_idx(jnp.minimum(ci + 2, NCHUNK - 1), q2)

        start_idx(0, 0)
        start_idx(1, 1)
        pltpu.sync_copy(z_hbm.at[pl.ds(rstart, ROW_COPY)],
                        acc_sh.at[pl.ds(rstart, ROW_COPY)])
        plsc.subcore_barrier()
        wait_idx(0)
        start_gather(0, 0)
        do_chunk(0, 0, wait_prev_scatter=False)
        do_chunk(1, 1)
        do_chunk(2, 2)
        do_chunk(3, 3)

        def quad(t, carry):
            ci = 4 * t + 4
            do_chunk(ci, 0)
            do_chunk(ci + 1, 1)
            do_chunk(ci + 2, 2)
            do_chunk(ci + 3, 3)
            return carry
        lax.fori_loop(0, (NCHUNK - 5) // 4, quad, 0)

        wait_idx(1)
        wait_rows_bytes(ssems[1], 1)
        do_chunk(NCHUNK - 1, 0, issue_next=False)
        wait_rows_bytes(ssems[0], 0)

        plsc.subcore_barrier()
        pltpu.sync_copy(acc_sh.at[pl.ds(rstart, ROW_COPY)],
                        out_hbm.at[cid, pl.ds(rstart, ROW_COPY)])

    return k(h, packed, vals3, zeros)


_BM = 1000


def _mm_in(x, w_t, b):
    def body(x_ref, w_ref, b_ref, o_ref):
        o_ref[...] = jnp.dot(x_ref[...], w_ref[...],
                             preferred_element_type=jnp.float32) + b_ref[...]
    return pl.pallas_call(
        body,
        grid=(N // _BM,),
        in_specs=[pl.BlockSpec((_BM, F), lambda i: (i, 0)),
                  pl.BlockSpec((F, F), lambda i: (0, 0)),
                  pl.BlockSpec((1, F), lambda i: (0, 0))],
        out_specs=pl.BlockSpec((_BM, F), lambda i: (i, 0)),
        out_shape=jax.ShapeDtypeStruct((N, F), jnp.float32),
    )(x, w_t, b.reshape(1, F))


def _combine_scale(parts, scal):
    def body(s_ref, p_ref, o_ref):
        s = p_ref[0] + p_ref[1]
        o_ref[...] = jnp.where(s > 0, s, (jnp.exp(s) - 1.0)) * s_ref[0]
    return pl.pallas_call(
        body,
        grid=(N // _BM,),
        in_specs=[pl.BlockSpec(memory_space=pltpu.SMEM),
                  pl.BlockSpec((NC, _BM, F), lambda i: (0, i, 0))],
        out_specs=pl.BlockSpec((_BM, F), lambda i: (i, 0)),
        out_shape=jax.ShapeDtypeStruct((N, F), jnp.float32),
    )(scal, parts)


def _combine_mm_out(parts, w_t, b):
    def body(p_ref, w_ref, b_ref, o_ref):
        s = p_ref[0] + p_ref[1]
        h = jnp.where(s > 0, s, (jnp.exp(s) - 1.0))
        o_ref[...] = jnp.dot(h, w_ref[...],
                             preferred_element_type=jnp.float32) + b_ref[...]
    return pl.pallas_call(
        body,
        grid=(N // _BM,),
        in_specs=[pl.BlockSpec((NC, _BM, F), lambda i: (0, i, 0)),
                  pl.BlockSpec((F, F), lambda i: (0, 0)),
                  pl.BlockSpec((1, F), lambda i: (0, 0))],
        out_specs=pl.BlockSpec((_BM, F), lambda i: (i, 0)),
        out_shape=jax.ShapeDtypeStruct((N, F), jnp.float32),
    )(parts, w_t, b.reshape(1, F))


def kernel(x, edge_index, edge_vals, W1, b1, scalar, Wout, bout):
    src3 = edge_index[1].reshape(NW, NCHUNK, 1, K)
    dst3 = edge_index[0].reshape(NW, NCHUNK, 1, K)
    packed = jnp.concatenate([src3, dst3], axis=2)
    vals3 = edge_vals.reshape(NW, NCHUNK, K)
    zeros = jnp.zeros((N, F), jnp.float32)

    h = _mm_in(x, W1.T, b1)
    parts = _spmm_partials(h, packed, vals3, zeros)
    for _ in range(2):
        g = _combine_scale(parts, scalar)
        parts = _spmm_partials(g, packed, vals3, zeros)
    return _combine_mm_out(parts, Wout.T, bout)
